# speedup vs baseline: 1.0783x; 1.0783x over previous
import functools

import jax
import jax.numpy as jnp
from jax import lax
from jax.experimental import pallas as pl
from jax.experimental.pallas import tpu as pltpu
from jax.experimental.pallas import tpu_sc as plsc

_NC = 2
_NS = 16
_NW = _NC * _NS


@functools.partial(jax.jit, static_argnames=("b_per_w", "chunk"))
def _sc_gather(x, idx, *, b_per_w, chunk):
    n_chunks = b_per_w // chunk
    d = x.shape[1]
    mesh = plsc.VectorSubcoreMesh(core_axis_name="c", subcore_axis_name="s")

    @functools.partial(
        pl.kernel,
        mesh=mesh,
        out_type=jax.ShapeDtypeStruct((b_per_w * _NW, d), jnp.float32),
        scratch_types=[
            pltpu.VMEM((b_per_w,), jnp.int32),
            pltpu.VMEM((chunk, d), jnp.float32),
            pltpu.VMEM((chunk, d), jnp.float32),
            pltpu.SemaphoreType.DMA,
            pltpu.SemaphoreType.DMA,
        ],
    )
    def k(table_hbm, idx_hbm, out_hbm, idx_v, buf0, buf1, sem0, sem1):
        wid = lax.axis_index("s") * _NC + lax.axis_index("c")
        base = wid * b_per_w
        pltpu.sync_copy(idx_hbm.at[pl.ds(base, b_per_w)], idx_v)

        bufs = (buf0, buf1)
        sems = (sem0, sem1)
        copies = [None, None]
        copies[0] = pltpu.async_copy(
            table_hbm.at[idx_v.at[pl.ds(0, chunk)]], bufs[0], sems[0]
        )
        for g in range(n_chunks):
            cur = g % 2
            nxt = (g + 1) % 2
            if g + 1 < n_chunks:
                copies[nxt] = pltpu.async_copy(
                    table_hbm.at[idx_v.at[pl.ds((g + 1) * chunk, chunk)]],
                    bufs[nxt],
                    sems[nxt],
                )
            copies[cur].wait()
            pltpu.sync_copy(bufs[cur], out_hbm.at[pl.ds(base + g * chunk, chunk)])

    return k(x, idx)


def kernel(x, coarse_idx):
    b = coarse_idx.shape[0]
    b_per_w = -(-b // (_NW * 8)) * 8
    idx = jnp.zeros((b_per_w * _NW,), jnp.int32).at[:b].set(
        coarse_idx.astype(jnp.int32))
    out = _sc_gather(x, idx, b_per_w=b_per_w, chunk=112)
    return out[:b]


# speedup vs baseline: 1.5218x; 1.4113x over previous
import functools

import jax
import jax.numpy as jnp
from jax import lax
from jax.experimental import pallas as pl
from jax.experimental.pallas import tpu as pltpu
from jax.experimental.pallas import tpu_sc as plsc

_NC = 2
_NS = 16
_NW = _NC * _NS


@functools.partial(jax.jit, static_argnames=("b", "b_per_w", "chunk"))
def _sc_gather(x, idx, *, b, b_per_w, chunk):
    n_chunks = b_per_w // chunk
    d = x.shape[1]
    tail = b - (_NW - 1) * b_per_w - (n_chunks - 1) * chunk
    assert 0 < tail <= chunk and tail % 8 == 0
    mesh = plsc.VectorSubcoreMesh(core_axis_name="c", subcore_axis_name="s")

    @functools.partial(
        pl.kernel,
        mesh=mesh,
        out_type=jax.ShapeDtypeStruct((b, d), jnp.float32),
        scratch_types=[
            pltpu.VMEM((b_per_w,), jnp.int32),
            pltpu.VMEM((chunk, d), jnp.float32),
            pltpu.VMEM((chunk, d), jnp.float32),
            pltpu.SemaphoreType.DMA,
            pltpu.SemaphoreType.DMA,
        ],
    )
    def k(table_hbm, idx_hbm, out_hbm, idx_v, buf0, buf1, sem0, sem1):
        wid = lax.axis_index("s") * _NC + lax.axis_index("c")
        base = wid * b_per_w
        pltpu.sync_copy(idx_hbm.at[pl.ds(base, b_per_w)], idx_v)

        bufs = (buf0, buf1)
        sems = (sem0, sem1)
        copies = [None, None]
        copies[0] = pltpu.async_copy(
            table_hbm.at[idx_v.at[pl.ds(0, chunk)]], bufs[0], sems[0]
        )
        for g in range(n_chunks):
            cur = g % 2
            nxt = (g + 1) % 2
            if g + 1 < n_chunks:
                copies[nxt] = pltpu.async_copy(
                    table_hbm.at[idx_v.at[pl.ds((g + 1) * chunk, chunk)]],
                    bufs[nxt],
                    sems[nxt],
                )
            copies[cur].wait()
            if g + 1 < n_chunks:
                pltpu.sync_copy(
                    bufs[cur], out_hbm.at[pl.ds(base + g * chunk, chunk)]
                )
            else:
                full = base + b_per_w <= b

                @pl.when(full)
                def _():
                    pltpu.sync_copy(
                        bufs[cur], out_hbm.at[pl.ds(base + g * chunk, chunk)]
                    )

                @pl.when(jnp.logical_not(full))
                def _():
                    pltpu.sync_copy(
                        bufs[cur].at[pl.ds(0, tail)],
                        out_hbm.at[pl.ds(base + g * chunk, tail)],
                    )

    return k(x, idx)


def kernel(x, coarse_idx):
    b = coarse_idx.shape[0]
    b_per_w = -(-b // (_NW * 8)) * 8
    idx = jnp.zeros((b_per_w * _NW,), jnp.int32).at[:b].set(
        coarse_idx.astype(jnp.int32))
    return _sc_gather(x, idx, b=b, b_per_w=b_per_w, chunk=112)


# speedup vs baseline: 1.5776x; 1.0367x over previous
import functools

import jax
import jax.numpy as jnp
from jax import lax
from jax.experimental import pallas as pl
from jax.experimental.pallas import tpu as pltpu
from jax.experimental.pallas import tpu_sc as plsc

_NC = 2
_NS = 16
_NW = _NC * _NS


@functools.partial(jax.jit, static_argnames=("b", "b_per_w", "chunk"))
def _sc_gather(x, idx, *, b, b_per_w, chunk):
    sizes = [chunk] * (b_per_w // chunk)
    if b_per_w % chunk:
        sizes.append(b_per_w % chunk)
    offs = [sum(sizes[:g]) for g in range(len(sizes))]
    n_chunks = len(sizes)
    d = x.shape[1]
    last_span = b - (_NW - 1) * b_per_w
    last_len = [min(max(last_span - offs[g], 0), sizes[g]) for g in range(n_chunks)]
    assert 0 < last_span <= b_per_w and all(l % 8 == 0 for l in last_len)
    mesh = plsc.VectorSubcoreMesh(core_axis_name="c", subcore_axis_name="s")

    @functools.partial(
        pl.kernel,
        mesh=mesh,
        out_type=jax.ShapeDtypeStruct((b, d), jnp.float32),
        scratch_types=[
            pltpu.VMEM((b_per_w,), jnp.int32),
            pltpu.VMEM((chunk, d), jnp.float32),
            pltpu.VMEM((chunk, d), jnp.float32),
            pltpu.SemaphoreType.DMA,
            pltpu.SemaphoreType.DMA,
        ],
    )
    def k(table_hbm, idx_hbm, out_hbm, idx_v, buf0, buf1, sem0, sem1):
        wid = lax.axis_index("s") * _NC + lax.axis_index("c")
        base = wid * b_per_w
        pltpu.sync_copy(idx_hbm.at[pl.ds(base, b_per_w)], idx_v)

        bufs = (buf0, buf1)
        sems = (sem0, sem1)
        copies = [None, None]
        copies[0] = pltpu.async_copy(
            table_hbm.at[idx_v.at[pl.ds(0, sizes[0])]],
            bufs[0].at[pl.ds(0, sizes[0])],
            sems[0],
        )
        for g in range(n_chunks):
            cur = g % 2
            nxt = (g + 1) % 2
            if g + 1 < n_chunks:
                copies[nxt] = pltpu.async_copy(
                    table_hbm.at[idx_v.at[pl.ds(offs[g + 1], sizes[g + 1])]],
                    bufs[nxt].at[pl.ds(0, sizes[g + 1])],
                    sems[nxt],
                )
            copies[cur].wait()
            if last_len[g] == sizes[g]:
                pltpu.sync_copy(
                    bufs[cur].at[pl.ds(0, sizes[g])],
                    out_hbm.at[pl.ds(base + offs[g], sizes[g])],
                )
            else:
                full = base + b_per_w <= b

                @pl.when(full)
                def _():
                    pltpu.sync_copy(
                        bufs[cur].at[pl.ds(0, sizes[g])],
                        out_hbm.at[pl.ds(base + offs[g], sizes[g])],
                    )

                if last_len[g] > 0:
                    lw = last_len[g]

                    @pl.when(jnp.logical_not(full))
                    def _():
                        pltpu.sync_copy(
                            bufs[cur].at[pl.ds(0, lw)],
                            out_hbm.at[pl.ds(base + offs[g], lw)],
                        )

    return k(x, idx)


def kernel(x, coarse_idx):
    b = coarse_idx.shape[0]
    b_per_w = -(-b // (_NW * 8)) * 8
    idx = jnp.zeros((b_per_w * _NW,), jnp.int32).at[:b].set(
        coarse_idx.astype(jnp.int32))
    return _sc_gather(x, idx, b=b, b_per_w=b_per_w, chunk=240)
